# Initial kernel scaffold; baseline (speedup 1.0000x reference)
import jax
import jax.numpy as jnp
from jax.experimental import pallas as pl


def kernel(x, codebook_w, enc_w, enc_b, cm_w1, cm_b1, cm_g1, cm_be1, cm_w2, cm_b2, cm_g2, cm_be2, cm_w3, cm_b3, dec_w1, dec_b1, dec_w2, dec_b2, dec_w3, dec_b3):
    raise NotImplementedError("write your pallas kernel here")


# speedup vs baseline: 1.7386x; 1.7386x over previous
import functools

import jax
import jax.numpy as jnp
from jax import lax
from jax.experimental import pallas as pl
from jax.experimental.pallas import tpu as pltpu
from jax.experimental.pallas import tpu_sc as plsc


def _ln(h, g, b):
    m = jnp.mean(h, axis=-1, keepdims=True)
    v = jnp.var(h, axis=-1, keepdims=True)
    return (h - m) / jnp.sqrt(v + 1e-5) * g + b


def _silu(h):
    return h * jax.nn.sigmoid(h)


def _tables_body(cb_ref, cm_w1_ref, cm_b1_ref, cm_g1_ref, cm_be1_ref,
                 cm_w2_ref, cm_b2_ref, cm_g2_ref, cm_be2_ref,
                 cm_w3_ref, cm_b3_ref, enc_w_ref, enc_b_ref,
                 dec_w1_ref, dec_b1_ref, dec_w2_ref, dec_b2_ref,
                 dec_w3_ref, dec_b3_ref,
                 lc_ref, te_ref, td_ref, c2_ref):
    cb = cb_ref[...]
    h = jnp.dot(cb, cm_w1_ref[...], preferred_element_type=jnp.float32)
    h = _silu(_ln(h + cm_b1_ref[...], cm_g1_ref[...], cm_be1_ref[...]))
    h = jnp.dot(h, cm_w2_ref[...], preferred_element_type=jnp.float32)
    h = _silu(_ln(h + cm_b2_ref[...], cm_g2_ref[...], cm_be2_ref[...]))
    lc = jnp.dot(h, cm_w3_ref[...], preferred_element_type=jnp.float32)
    lc = lc + cm_b3_ref[...]
    te = jnp.dot(lc, enc_w_ref[...], preferred_element_type=jnp.float32)
    te = te + enc_b_ref[...]
    hd = _silu(jnp.dot(te, dec_w1_ref[...], preferred_element_type=jnp.float32)
               + dec_b1_ref[...])
    hd = _silu(jnp.dot(hd, dec_w2_ref[...], preferred_element_type=jnp.float32)
               + dec_b2_ref[...])
    td = jnp.dot(hd, dec_w3_ref[...], preferred_element_type=jnp.float32)
    td = td + dec_b3_ref[...]
    lc_ref[...] = lc
    te_ref[...] = te
    td_ref[...] = td
    c2_ref[...] = jnp.sum(lc * lc, axis=1, keepdims=True)


def _encode_body(x_ref, enc_w_ref, enc_b_ref, lct_ref, c2_ref,
                 z_ref, tok_ref):
    x = x_ref[...]
    z_ref[...] = (jnp.dot(x, enc_w_ref[...], preferred_element_type=jnp.float32)
                  + enc_b_ref[...])
    m = jnp.dot(x, lct_ref[...], preferred_element_type=jnp.float32)
    d = (jnp.sum(x * x, axis=1, keepdims=True) + c2_ref[...]) - 2.0 * m
    v = d.shape[1]
    dmin = jnp.min(d, axis=1, keepdims=True)
    ids = lax.broadcasted_iota(jnp.int32, d.shape, 1)
    tok = jnp.min(jnp.where(d <= dmin, ids, jnp.int32(v)), axis=1)
    tok_ref[0, 0, :] = tok


def _make_sc_gather(n_tok, v, e):
    nc, ns = 2, 16
    nw = nc * ns
    tpw = n_tok // nw
    ch = 128
    nch = tpw // ch

    mesh = plsc.VectorSubcoreMesh(core_axis_name="c", subcore_axis_name="s",
                                  num_cores=nc, num_subcores=ns)

    @functools.partial(
        pl.kernel,
        out_type=(jax.ShapeDtypeStruct((n_tok, e), jnp.float32),
                  jax.ShapeDtypeStruct((n_tok * 3,), jnp.float32)),
        mesh=mesh,
        scratch_types=[
            pltpu.VMEM((tpw,), jnp.int32),
            pltpu.VMEM((ch, e), jnp.float32),
            pltpu.VMEM((ch, e), jnp.float32),
            pltpu.VMEM((v * 3,), jnp.float32),
            pltpu.VMEM((tpw * 3,), jnp.float32),
            pltpu.SemaphoreType.DMA,
            pltpu.SemaphoreType.DMA,
        ],
        compiler_params=pltpu.CompilerParams(needs_layout_passes=False),
    )
    def sc_gather(tok_hbm, te_hbm, td_hbm, zq_hbm, rec_hbm,
                  idx_v, g0, g1, tdv, recv, s0, s1):
        w = lax.axis_index("s") * nc + lax.axis_index("c")
        base = w * tpw
        pltpu.sync_copy(tok_hbm.at[pl.ds(base, tpw)], idx_v)
        pltpu.sync_copy(td_hbm, tdv)

        bufs = (g0, g1)
        sems = (s0, s1)
        handles = [None, None]
        handles[0] = pltpu.async_copy(
            te_hbm.at[idx_v.at[pl.ds(0, ch)]], g0, s0)
        for k in range(nch):
            b = k & 1
            if k + 1 < nch:
                handles[1 - b] = pltpu.async_copy(
                    te_hbm.at[idx_v.at[pl.ds((k + 1) * ch, ch)]],
                    bufs[1 - b], sems[1 - b])
            handles[b].wait()
            pltpu.sync_copy(bufs[b], zq_hbm.at[pl.ds(base + k * ch, ch)])

        def rec_group(g, carry):
            idx = idx_v[pl.ds(g * 16, 16)]
            f = idx * 3
            p = (g * 16 + lax.broadcasted_iota(jnp.int32, (16,), 0)) * 3
            plsc.store_scatter(recv, [p], plsc.load_gather(tdv, [f]))
            plsc.store_scatter(recv, [p + 1], plsc.load_gather(tdv, [f + 1]))
            plsc.store_scatter(recv, [p + 2], plsc.load_gather(tdv, [f + 2]))
            return carry

        lax.fori_loop(0, tpw // 16, rec_group, 0)
        pltpu.sync_copy(recv, rec_hbm.at[pl.ds(w * tpw * 3, tpw * 3)])

    return sc_gather


def kernel(x, codebook_w, enc_w, enc_b, cm_w1, cm_b1, cm_g1, cm_be1,
           cm_w2, cm_b2, cm_g2, cm_be2, cm_w3, cm_b3,
           dec_w1, dec_b1, dec_w2, dec_b2, dec_w3, dec_b3):
    b, in_dim = x.shape
    v, d = codebook_w.shape
    e = enc_w.shape[1]
    k_tok = in_dim // d
    n = b * k_tok

    x_flat = x.reshape(n, d)
    row = lambda a: a.reshape(1, -1)

    lc, te, td, c2 = pl.pallas_call(
        _tables_body,
        out_shape=(jax.ShapeDtypeStruct((v, d), jnp.float32),
                   jax.ShapeDtypeStruct((v, e), jnp.float32),
                   jax.ShapeDtypeStruct((v, d), jnp.float32),
                   jax.ShapeDtypeStruct((v, 1), jnp.float32)),
    )(codebook_w, cm_w1, row(cm_b1), row(cm_g1), row(cm_be1),
      cm_w2, row(cm_b2), row(cm_g2), row(cm_be2),
      cm_w3, row(cm_b3), enc_w, row(enc_b),
      dec_w1, row(dec_b1), dec_w2, row(dec_b2), dec_w3, row(dec_b3))

    lct = lc.T
    c2r = c2.reshape(1, v)

    grid = 32
    t = n // grid
    z, tok3 = pl.pallas_call(
        _encode_body,
        grid=(grid,),
        in_specs=[pl.BlockSpec((t, d), lambda i: (i, 0)),
                  pl.BlockSpec((d, e), lambda i: (0, 0)),
                  pl.BlockSpec((1, e), lambda i: (0, 0)),
                  pl.BlockSpec((d, v), lambda i: (0, 0)),
                  pl.BlockSpec((1, v), lambda i: (0, 0))],
        out_specs=(pl.BlockSpec((t, e), lambda i: (i, 0)),
                   pl.BlockSpec((1, 1, t), lambda i: (i, 0, 0))),
        out_shape=(jax.ShapeDtypeStruct((n, e), jnp.float32),
                   jax.ShapeDtypeStruct((grid, 1, t), jnp.int32)),
    )(x_flat, enc_w, row(enc_b), lct, c2r)

    tokens = tok3.reshape(n)
    zq, rec_flat = _make_sc_gather(n, v, e)(tokens, te, td.reshape(v * 3))

    return (z.reshape(b, k_tok, e),
            zq.reshape(b, k_tok, e),
            rec_flat.reshape(b, in_dim))
